# baseline (speedup 1.0000x reference)
import functools

import jax
import jax.numpy as jnp
from jax import lax
from jax.experimental import pallas as pl
from jax.experimental.pallas import tpu as pltpu
from jax.experimental.pallas import tpu_sc as plsc

_EMBED_DIM = 64
_BATCH = 16384
_BLK = 8
_ROW = 16
_BSZ = 4


def _gather_call(batch_2d, table):
    info = plsc.get_sparse_core_info()
    num_workers = info.num_cores * info.num_subcores
    b_per_w = _BATCH // num_workers
    n_rows = b_per_w // _ROW
    n_batches = _ROW // _BSZ
    mesh = plsc.VectorSubcoreMesh(core_axis_name="c", subcore_axis_name="s")

    @functools.partial(
        pl.kernel,
        mesh=mesh,
        out_type=jax.ShapeDtypeStruct((_BATCH, _EMBED_DIM), jnp.float32),
        scratch_types=[
            pltpu.VMEM((n_rows, _ROW), jnp.int32),
            pltpu.VMEM((2 * _BSZ, _BLK, _EMBED_DIM), jnp.float32),
            pltpu.VMEM((b_per_w, _EMBED_DIM), jnp.float32),
            pltpu.SemaphoreType.DMA,
            pltpu.SemaphoreType.DMA,
        ],
        compiler_params=pltpu.CompilerParams(use_tc_tiling_on_sc=True),
    )
    def gather_kernel(idx_hbm, table_hbm, out_hbm, idx_v, blocks, slab, semA, semB):
        wid = lax.axis_index("s") * info.num_cores + lax.axis_index("c")
        base = wid * b_per_w
        pltpu.sync_copy(idx_hbm.at[pl.ds(wid * n_rows, n_rows), :], idx_v)

        sems = (semA, semB)

        def fire(q, slot, sem):
            pltpu.async_copy(
                table_hbm.at[pl.ds(pl.multiple_of(q * _BLK, _BLK), _BLK), :],
                blocks.at[slot],
                sem,
            )

        def wait_batch(sem):
            for _ in range(_BSZ):
                pltpu.make_async_copy(
                    table_hbm.at[pl.ds(0, _BLK), :], blocks.at[0], sem
                ).wait()

        def extract(slot, r, p):
            slab[p, :] = blocks[slot, r, :]

        def body(h, carry):
            vec = idx_v[h, :]
            qs = [vec[l] >> 3 for l in range(_ROW)]
            rs = [vec[l] & 7 for l in range(_ROW)]

            for j in range(_BSZ):
                fire(qs[j], j, sems[0])
            for b in range(1, n_batches):
                for j in range(_BSZ):
                    fire(qs[b * _BSZ + j], (b % 2) * _BSZ + j, sems[b % 2])
                wait_batch(sems[(b - 1) % 2])
                for j in range(_BSZ):
                    l = (b - 1) * _BSZ + j
                    extract(((b - 1) % 2) * _BSZ + j, rs[l], h * _ROW + l)
            wait_batch(sems[(n_batches - 1) % 2])
            for j in range(_BSZ):
                l = (n_batches - 1) * _BSZ + j
                extract(((n_batches - 1) % 2) * _BSZ + j, rs[l], h * _ROW + l)
            return carry

        lax.fori_loop(0, n_rows, body, 0)
        pltpu.sync_copy(slab, out_hbm.at[pl.ds(base, b_per_w), :])

    return gather_kernel(batch_2d, table)


def kernel(batch, emb_weight):
    num_idx_rows = _BATCH // _ROW
    batch_2d = batch.astype(jnp.int32).reshape(num_idx_rows, _ROW)
    return _gather_call(batch_2d, emb_weight)


# speedup vs baseline: 1.0363x; 1.0363x over previous
import functools

import jax
import jax.numpy as jnp
from jax import lax
from jax.experimental import pallas as pl
from jax.experimental.pallas import tpu as pltpu
from jax.experimental.pallas import tpu_sc as plsc

_EMBED_DIM = 64
_BATCH = 16384
_BLK = 8
_ROW = 16
_BSZ = 8


def _gather_call(batch_2d, table):
    info = plsc.get_sparse_core_info()
    num_workers = info.num_cores * info.num_subcores
    b_per_w = _BATCH // num_workers
    n_rows = b_per_w // _ROW
    n_batches = _ROW // _BSZ
    mesh = plsc.VectorSubcoreMesh(core_axis_name="c", subcore_axis_name="s")

    @functools.partial(
        pl.kernel,
        mesh=mesh,
        out_type=jax.ShapeDtypeStruct((_BATCH, _EMBED_DIM), jnp.float32),
        scratch_types=[
            pltpu.VMEM((n_rows, _ROW), jnp.int32),
            pltpu.VMEM((2 * _BSZ, _BLK, _EMBED_DIM), jnp.float32),
            pltpu.VMEM((b_per_w, _EMBED_DIM), jnp.float32),
            pltpu.SemaphoreType.DMA,
            pltpu.SemaphoreType.DMA,
        ],
        compiler_params=pltpu.CompilerParams(use_tc_tiling_on_sc=True),
    )
    def gather_kernel(idx_hbm, table_hbm, out_hbm, idx_v, blocks, slab, semA, semB):
        wid = lax.axis_index("s") * info.num_cores + lax.axis_index("c")
        base = wid * b_per_w
        pltpu.sync_copy(idx_hbm.at[pl.ds(wid * n_rows, n_rows), :], idx_v)

        sems = (semA, semB)

        def fire(q, slot, sem):
            pltpu.async_copy(
                table_hbm.at[pl.ds(pl.multiple_of(q * _BLK, _BLK), _BLK), :],
                blocks.at[slot],
                sem,
            )

        def wait_batch(sem):
            for _ in range(_BSZ):
                pltpu.make_async_copy(
                    table_hbm.at[pl.ds(0, _BLK), :], blocks.at[0], sem
                ).wait()

        def extract(slot, r, p):
            slab[p, :] = blocks[slot, r, :]

        def body(h, carry):
            vec = idx_v[h, :]
            qs = [vec[l] >> 3 for l in range(_ROW)]
            rs = [vec[l] & 7 for l in range(_ROW)]

            for j in range(_BSZ):
                fire(qs[j], j, sems[0])
            for b in range(1, n_batches):
                for j in range(_BSZ):
                    fire(qs[b * _BSZ + j], (b % 2) * _BSZ + j, sems[b % 2])
                wait_batch(sems[(b - 1) % 2])
                for j in range(_BSZ):
                    l = (b - 1) * _BSZ + j
                    extract(((b - 1) % 2) * _BSZ + j, rs[l], h * _ROW + l)
            wait_batch(sems[(n_batches - 1) % 2])
            for j in range(_BSZ):
                l = (n_batches - 1) * _BSZ + j
                extract(((n_batches - 1) % 2) * _BSZ + j, rs[l], h * _ROW + l)
            return carry

        lax.fori_loop(0, n_rows, body, 0)
        pltpu.sync_copy(slab, out_hbm.at[pl.ds(base, b_per_w), :])

    return gather_kernel(batch_2d, table)


def kernel(batch, emb_weight):
    num_idx_rows = _BATCH // _ROW
    batch_2d = batch.astype(jnp.int32).reshape(num_idx_rows, _ROW)
    return _gather_call(batch_2d, emb_weight)


# speedup vs baseline: 1.0408x; 1.0044x over previous
import functools

import jax
import jax.numpy as jnp
from jax import lax
from jax.experimental import pallas as pl
from jax.experimental.pallas import tpu as pltpu
from jax.experimental.pallas import tpu_sc as plsc

_EMBED_DIM = 64
_BATCH = 16384
_BLK = 8
_ROW = 32
_BSZ = 8


def _gather_call(batch_2d, table):
    info = plsc.get_sparse_core_info()
    num_workers = info.num_cores * info.num_subcores
    b_per_w = _BATCH // num_workers
    n_rows = b_per_w // _ROW
    n_batches = _ROW // _BSZ
    mesh = plsc.VectorSubcoreMesh(core_axis_name="c", subcore_axis_name="s")

    @functools.partial(
        pl.kernel,
        mesh=mesh,
        out_type=jax.ShapeDtypeStruct((_BATCH, _EMBED_DIM), jnp.float32),
        scratch_types=[
            pltpu.VMEM((n_rows, _ROW), jnp.int32),
            pltpu.VMEM((2 * _BSZ, _BLK, _EMBED_DIM), jnp.float32),
            pltpu.VMEM((b_per_w, _EMBED_DIM), jnp.float32),
            pltpu.SemaphoreType.DMA,
            pltpu.SemaphoreType.DMA,
        ],
        compiler_params=pltpu.CompilerParams(use_tc_tiling_on_sc=True),
    )
    def gather_kernel(idx_hbm, table_hbm, out_hbm, idx_v, blocks, slab, semA, semB):
        wid = lax.axis_index("s") * info.num_cores + lax.axis_index("c")
        base = wid * b_per_w
        pltpu.sync_copy(idx_hbm.at[pl.ds(wid * n_rows, n_rows), :], idx_v)

        sems = (semA, semB)

        def fire(q, slot, sem):
            pltpu.async_copy(
                table_hbm.at[pl.ds(pl.multiple_of(q * _BLK, _BLK), _BLK), :],
                blocks.at[slot],
                sem,
            )

        def wait_batch(sem):
            for _ in range(_BSZ):
                pltpu.make_async_copy(
                    table_hbm.at[pl.ds(0, _BLK), :], blocks.at[0], sem
                ).wait()

        def extract(slot, r, p):
            slab[p, :] = blocks[slot, r, :]

        def body(h, carry):
            vec = idx_v[h, :]
            qs = [vec[l] >> 3 for l in range(_ROW)]
            rs = [vec[l] & 7 for l in range(_ROW)]

            for j in range(_BSZ):
                fire(qs[j], j, sems[0])
            for b in range(1, n_batches):
                for j in range(_BSZ):
                    fire(qs[b * _BSZ + j], (b % 2) * _BSZ + j, sems[b % 2])
                wait_batch(sems[(b - 1) % 2])
                for j in range(_BSZ):
                    l = (b - 1) * _BSZ + j
                    extract(((b - 1) % 2) * _BSZ + j, rs[l], h * _ROW + l)
            wait_batch(sems[(n_batches - 1) % 2])
            for j in range(_BSZ):
                l = (n_batches - 1) * _BSZ + j
                extract(((n_batches - 1) % 2) * _BSZ + j, rs[l], h * _ROW + l)
            return carry

        lax.fori_loop(0, n_rows, body, 0)
        pltpu.sync_copy(slab, out_hbm.at[pl.ds(base, b_per_w), :])

    return gather_kernel(batch_2d, table)


def kernel(batch, emb_weight):
    num_idx_rows = _BATCH // _ROW
    batch_2d = batch.astype(jnp.int32).reshape(num_idx_rows, _ROW)
    return _gather_call(batch_2d, emb_weight)
